# baseline (speedup 1.0000x reference)
import functools

import jax
import jax.numpy as jnp
from jax import lax
from jax.experimental import pallas as pl
from jax.experimental.pallas import tpu as pltpu
from jax.experimental.pallas import tpu_sc as plsc

_EPS = 1e-5
_LANES = 16


def _rsqrt(x):
    bits = lax.bitcast_convert_type(x, jnp.int32)
    y = lax.bitcast_convert_type(jnp.int32(0x5F3759DF) - (bits >> 1), jnp.float32)
    for _ in range(3):
        y = y * (1.5 - 0.5 * x * y * y)
    return y


def _tree_sum(vs):
    vs = list(vs)
    while len(vs) > 1:
        nxt = [a + b for a, b in zip(vs[0::2], vs[1::2])]
        if len(vs) % 2:
            nxt.append(vs[-1])
        vs = nxt
    return vs[0]


@functools.lru_cache(maxsize=None)
def _build(n_rows, vocab, d):
    info = plsc.get_sparse_core_info()
    nc, ns = info.num_cores, info.num_subcores
    nw = nc * ns
    per_w = n_rows // nw
    chunk_rows = 1024
    n_chunks = per_w // chunk_rows
    kb = chunk_rows // 128
    groups = chunk_rows // _LANES
    kb_all = per_w // 128

    mesh = plsc.VectorSubcoreMesh(core_axis_name="c", subcore_axis_name="s")

    @functools.partial(
        pl.kernel,
        out_type=jax.ShapeDtypeStruct((n_rows, d), jnp.float32),
        mesh=mesh,
        scratch_types=[
            pltpu.VMEM((kb_all, 128), jnp.int32),
            pltpu.VMEM((chunk_rows, d), jnp.float32),
            pltpu.VMEM((chunk_rows, d), jnp.float32),
            pltpu.VMEM((chunk_rows, d), jnp.float32),
            pltpu.VMEM((d, _LANES), jnp.float32),
            pltpu.VMEM((d, _LANES), jnp.float32),
            pltpu.SemaphoreType.DMA,
            pltpu.SemaphoreType.DMA,
            pltpu.SemaphoreType.DMA,
            pltpu.SemaphoreType.DMA,
            pltpu.SemaphoreType.DMA,
            pltpu.SemaphoreType.DMA,
        ],
        compiler_params=pltpu.CompilerParams(
            needs_layout_passes=False, use_tc_tiling_on_sc=False),
    )
    def sc_kernel(idx_hbm, table_hbm, gs_hbm, gb_hbm, out_hbm,
                  idx_v, rows0, rows1, rows2, gs_v, gb_v,
                  g0, g1, g2, o0, o1, o2):
        rows = (rows0, rows1, rows2)
        gsem = (g0, g1, g2)
        osem = (o0, o1, o2)
        wid = lax.axis_index("s") * nc + lax.axis_index("c")
        pltpu.sync_copy(gs_hbm, gs_v)
        pltpu.sync_copy(gb_hbm, gb_v)
        pltpu.sync_copy(
            idx_hbm.at[pl.ds(pl.multiple_of(wid * kb_all, 8), kb_all)], idx_v)
        lane = jnp.arange(_LANES, dtype=jnp.int32)
        row_base = pl.multiple_of(wid * per_w, chunk_rows)

        def fire(c, q):
            for k in range(kb):
                pltpu.async_copy(
                    table_hbm.at[idx_v.at[c * kb + k]],
                    rows[q].at[pl.ds(k * 128, 128)], gsem[q])

        def drain_gather(q):
            pltpu.make_async_copy(
                out_hbm.at[pl.ds(0, chunk_rows)], rows[q], gsem[q]).wait()

        def drain_out(q):
            pltpu.make_async_copy(
                out_hbm.at[pl.ds(0, chunk_rows)], rows[q], osem[q]).wait()

        def compute(buf):
            def group(g, c2):
                rid = g * _LANES + lane
                cols = [
                    plsc.load_gather(
                        buf, [rid, jnp.full((_LANES,), j, jnp.int32)])
                    for j in range(d)
                ]
                mean = _tree_sum(cols) * (1.0 / d)
                xms = [x - mean for x in cols]
                var = _tree_sum([x * x for x in xms]) * (1.0 / d)
                inv = _rsqrt(var + _EPS)
                for j in range(d):
                    o = xms[j] * (inv * gs_v[j]) + gb_v[j]
                    plsc.store_scatter(
                        buf, [rid, jnp.full((_LANES,), j, jnp.int32)], o)
                return c2
            lax.fori_loop(0, groups, group, 0)

        def step(c, p):
            q = (p + 1) % 3

            @pl.when(c >= 2)
            def _():
                drain_out(q)

            fire(c + 1, q)
            drain_gather(p)
            compute(rows[p])
            pltpu.async_copy(
                rows[p],
                out_hbm.at[pl.ds(
                    pl.multiple_of(row_base + (c * chunk_rows), chunk_rows),
                    chunk_rows)],
                osem[p])

        fire(0, 0)

        def body(i, carry):
            c0 = i * 3
            step(c0, 0)
            step(c0 + 1, 1)
            step(c0 + 2, 2)
            return carry

        lax.fori_loop(0, (n_chunks - 1) // 3, body, 0)

        c_last = n_chunks - 1
        drain_gather(0)
        compute(rows0)
        pltpu.sync_copy(
            rows0,
            out_hbm.at[pl.ds(
                pl.multiple_of(row_base + c_last * chunk_rows, chunk_rows),
                chunk_rows)])
        drain_out(1)
        drain_out(2)

    return sc_kernel


def kernel(action_ids, table, gamma, beta):
    b, h = action_ids.shape
    vocab, d = table.shape
    n = b * h
    ids = action_ids.reshape(n // 128, 128).astype(jnp.int32)
    gs = jnp.broadcast_to(gamma.astype(jnp.float32)[:, None], (d, _LANES))
    gb = jnp.broadcast_to(beta.astype(jnp.float32)[:, None], (d, _LANES))
    out = _build(n, vocab, d)(ids, table, gs, gb)
    return out.reshape(b, h, d)


# speedup vs baseline: 1.9689x; 1.9689x over previous
import functools

import jax
import jax.numpy as jnp
from jax import lax
from jax.experimental import pallas as pl
from jax.experimental.pallas import tpu as pltpu
from jax.experimental.pallas import tpu_sc as plsc

_EPS = 1e-5
_LANES = 16


def _rsqrt(x):
    bits = lax.bitcast_convert_type(x, jnp.int32)
    y = lax.bitcast_convert_type(jnp.int32(0x5F3759DF) - (bits >> 1), jnp.float32)
    for _ in range(3):
        y = y * (1.5 - 0.5 * x * y * y)
    return y


def _tree_sum(vs):
    vs = list(vs)
    while len(vs) > 1:
        nxt = [a + b for a, b in zip(vs[0::2], vs[1::2])]
        if len(vs) % 2:
            nxt.append(vs[-1])
        vs = nxt
    return vs[0]


@functools.lru_cache(maxsize=None)
def _build(b_len, h_len, vocab, d):
    info = plsc.get_sparse_core_info()
    nc, ns = info.num_cores, info.num_subcores
    nw = nc * ns
    b_blk = b_len // nw
    n_chunks = h_len
    kb = b_blk // 128
    groups = b_blk // _LANES

    mesh = plsc.VectorSubcoreMesh(core_axis_name="c", subcore_axis_name="s")

    @functools.partial(
        pl.kernel,
        out_type=jax.ShapeDtypeStruct((h_len, d, b_len), jnp.float32),
        mesh=mesh,
        scratch_types=[
            pltpu.VMEM((b_blk,), jnp.int32),
            pltpu.VMEM((b_blk,), jnp.int32),
            pltpu.VMEM((b_blk, d), jnp.float32),
            pltpu.VMEM((b_blk, d), jnp.float32),
            pltpu.VMEM((d, b_blk), jnp.float32),
            pltpu.VMEM((d, b_blk), jnp.float32),
            pltpu.VMEM((d, _LANES), jnp.float32),
            pltpu.VMEM((d, _LANES), jnp.float32),
            pltpu.SemaphoreType.DMA,
            pltpu.SemaphoreType.DMA,
            pltpu.SemaphoreType.DMA,
            pltpu.SemaphoreType.DMA,
        ],
        compiler_params=pltpu.CompilerParams(
            needs_layout_passes=False, use_tc_tiling_on_sc=False),
    )
    def sc_kernel(ids_hbm, table_hbm, gs_hbm, gb_hbm, out_hbm,
                  idx0, idx1, rows0, rows1, ob0, ob1, gs_v, gb_v,
                  gsem0, gsem1, osem0, osem1):
        idxs = (idx0, idx1)
        rows = (rows0, rows1)
        obs = (ob0, ob1)
        gsems = (gsem0, gsem1)
        osems = (osem0, osem1)
        wid = lax.axis_index("s") * nc + lax.axis_index("c")
        b0 = pl.multiple_of(wid * b_blk, b_blk)
        pltpu.sync_copy(gs_hbm, gs_v)
        pltpu.sync_copy(gb_hbm, gb_v)
        lane = jnp.arange(_LANES, dtype=jnp.int32)

        def fire(c, q):
            pltpu.sync_copy(ids_hbm.at[c, pl.ds(b0, b_blk)], idxs[q])
            for k in range(kb):
                pltpu.async_copy(
                    table_hbm.at[idxs[q].at[pl.ds(k * 128, 128)]],
                    rows[q].at[pl.ds(k * 128, 128)], gsems[q])

        def drain_gather(p):
            pltpu.make_async_copy(
                table_hbm.at[pl.ds(0, b_blk)], rows[p], gsems[p]).wait()

        def drain_out(p):
            pltpu.make_async_copy(
                obs[p], out_hbm.at[0, :, pl.ds(0, b_blk)], osems[p]).wait()

        def compute(p):
            def group(g, cy):
                rid = g * _LANES + lane
                cols = [
                    plsc.load_gather(
                        rows[p], [rid, jnp.full((_LANES,), j, jnp.int32)])
                    for j in range(d)
                ]
                mean = _tree_sum(cols) * (1.0 / d)
                xms = [x - mean for x in cols]
                var = _tree_sum([x * x for x in xms]) * (1.0 / d)
                inv = _rsqrt(var + _EPS)
                for j in range(d):
                    obs[p][j, pl.ds(g * _LANES, _LANES)] = (
                        xms[j] * (inv * gs_v[j]) + gb_v[j])
                return cy

            lax.fori_loop(0, groups, group, 0)

        def step(c, p):
            q = 1 - p

            @pl.when(c + 1 < n_chunks)
            def _():
                fire(c + 1, q)

            drain_gather(p)

            @pl.when(c >= 2)
            def _():
                drain_out(p)

            compute(p)
            pltpu.async_copy(
                obs[p], out_hbm.at[c, :, pl.ds(b0, b_blk)], osems[p])

        fire(0, 0)

        def body(i, carry):
            step(i * 2, 0)
            step(i * 2 + 1, 1)
            return carry

        lax.fori_loop(0, n_chunks // 2, body, 0)
        drain_out(0)
        drain_out(1)

    return sc_kernel


def kernel(action_ids, table, gamma, beta):
    b, h = action_ids.shape
    vocab, d = table.shape
    ids_t = jnp.asarray(action_ids, jnp.int32).T
    gs = jnp.broadcast_to(gamma.astype(jnp.float32)[:, None], (d, _LANES))
    gb = jnp.broadcast_to(beta.astype(jnp.float32)[:, None], (d, _LANES))
    out_t = _build(b, h, vocab, d)(ids_t, table, gs, gb)
    return out_t.transpose(2, 0, 1)


# speedup vs baseline: 1.9706x; 1.0009x over previous
import functools

import jax
import jax.numpy as jnp
from jax import lax
from jax.experimental import pallas as pl
from jax.experimental.pallas import tpu as pltpu
from jax.experimental.pallas import tpu_sc as plsc

_EPS = 1e-5
_LANES = 16
_NBUF = 3


def _rsqrt(x):
    bits = lax.bitcast_convert_type(x, jnp.int32)
    y = lax.bitcast_convert_type(jnp.int32(0x5F3759DF) - (bits >> 1), jnp.float32)
    for _ in range(3):
        y = y * (1.5 - 0.5 * x * y * y)
    return y


def _tree_sum(vs):
    vs = list(vs)
    while len(vs) > 1:
        nxt = [a + b for a, b in zip(vs[0::2], vs[1::2])]
        if len(vs) % 2:
            nxt.append(vs[-1])
        vs = nxt
    return vs[0]


@functools.lru_cache(maxsize=None)
def _build(b_len, h_len, vocab, d):
    info = plsc.get_sparse_core_info()
    nc, ns = info.num_cores, info.num_subcores
    nw = nc * ns
    b_blk = b_len // nw
    n_chunks = h_len
    kb = b_blk // 128
    groups = b_blk // _LANES
    n_iters = (n_chunks + _NBUF - 1) // _NBUF

    mesh = plsc.VectorSubcoreMesh(core_axis_name="c", subcore_axis_name="s")

    @functools.partial(
        pl.kernel,
        out_type=jax.ShapeDtypeStruct((h_len, d, b_len), jnp.float32),
        mesh=mesh,
        scratch_types=(
            [pltpu.VMEM((b_blk,), jnp.int32)] * _NBUF
            + [pltpu.VMEM((b_blk, d), jnp.float32)] * _NBUF
            + [pltpu.VMEM((d, b_blk), jnp.float32)] * _NBUF
            + [pltpu.VMEM((d, _LANES), jnp.float32)] * 2
            + [pltpu.SemaphoreType.DMA] * (2 * _NBUF)
        ),
        compiler_params=pltpu.CompilerParams(
            needs_layout_passes=False, use_tc_tiling_on_sc=False),
    )
    def sc_kernel(ids_hbm, table_hbm, gs_hbm, gb_hbm, out_hbm, *refs):
        idxs = refs[0:_NBUF]
        rows = refs[_NBUF:2 * _NBUF]
        obs = refs[2 * _NBUF:3 * _NBUF]
        gs_v, gb_v = refs[3 * _NBUF:3 * _NBUF + 2]
        gsems = refs[3 * _NBUF + 2:4 * _NBUF + 2]
        osems = refs[4 * _NBUF + 2:5 * _NBUF + 2]
        wid = lax.axis_index("s") * nc + lax.axis_index("c")
        b0 = pl.multiple_of(wid * b_blk, b_blk)
        pltpu.sync_copy(gs_hbm, gs_v)
        pltpu.sync_copy(gb_hbm, gb_v)
        lane = jnp.arange(_LANES, dtype=jnp.int32)

        def fire(c, q):
            pltpu.sync_copy(ids_hbm.at[c, pl.ds(b0, b_blk)], idxs[q])
            for k in range(kb):
                pltpu.async_copy(
                    table_hbm.at[idxs[q].at[pl.ds(k * 128, 128)]],
                    rows[q].at[pl.ds(k * 128, 128)], gsems[q])

        def drain_gather(p):
            pltpu.make_async_copy(
                table_hbm.at[pl.ds(0, b_blk)], rows[p], gsems[p]).wait()

        def drain_out(p):
            pltpu.make_async_copy(
                obs[p], out_hbm.at[0, :, pl.ds(0, b_blk)], osems[p]).wait()

        def compute(p):
            def group(g, cy):
                rid = g * _LANES + lane
                cols = [
                    plsc.load_gather(
                        rows[p], [rid, jnp.full((_LANES,), j, jnp.int32)])
                    for j in range(d)
                ]
                mean = _tree_sum(cols) * (1.0 / d)
                xms = [x - mean for x in cols]
                var = _tree_sum([x * x for x in xms]) * (1.0 / d)
                inv = _rsqrt(var + _EPS)
                for j in range(d):
                    obs[p][j, pl.ds(g * _LANES, _LANES)] = (
                        xms[j] * (inv * gs_v[j]) + gb_v[j])
                return cy

            lax.fori_loop(0, groups, group, 0)

        def step(c, p):
            @pl.when(c < n_chunks)
            def _():
                @pl.when(c + 2 < n_chunks)
                def _():
                    fire(c + 2, (p + 2) % _NBUF)

                drain_gather(p)

                @pl.when(c >= _NBUF)
                def _():
                    drain_out(p)

                compute(p)
                pltpu.async_copy(
                    obs[p], out_hbm.at[c, :, pl.ds(b0, b_blk)], osems[p])

        fire(0, 0)
        fire(1, 1)

        def body(i, carry):
            for j in range(_NBUF):
                step(i * _NBUF + j, j)
            return carry

        lax.fori_loop(0, n_iters, body, 0)
        for p in range(_NBUF):
            drain_out(p)

    return sc_kernel


def kernel(action_ids, table, gamma, beta):
    b, h = action_ids.shape
    vocab, d = table.shape
    ids_t = jnp.asarray(action_ids, jnp.int32).T
    gs = jnp.broadcast_to(gamma.astype(jnp.float32)[:, None], (d, _LANES))
    gb = jnp.broadcast_to(beta.astype(jnp.float32)[:, None], (d, _LANES))
    out_t = _build(b, h, vocab, d)(ids_t, table, gs, gb)
    return out_t.transpose(2, 0, 1)
